# Initial kernel scaffold; baseline (speedup 1.0000x reference)
import jax
import jax.numpy as jnp
from jax.experimental import pallas as pl


def kernel(tokens, task_ids, task_table, Wg, bg, We, be, Wu, bu):
    raise NotImplementedError("write your pallas kernel here")


# speedup vs baseline: 4.3621x; 4.3621x over previous
import jax
import jax.numpy as jnp
from jax import lax
from jax.experimental import pallas as pl

B, N, D, E, T, K = 2, 2048, 768, 8, 16, 2
BT = 512


def _gelu(x):
    return x * 0.5 * (1.0 + lax.erf(x * (2.0 ** -0.5)))


def _moe_body(x_ref, taskoh_ref, tt_ref, wg_ref, bg_ref, we_ref, be_ref,
              wu_ref, bu_ref, out_ref):
    x = x_ref[0]
    tvec_all = jnp.dot(taskoh_ref[...], tt_ref[...],
                       preferred_element_type=jnp.float32)
    brow = lax.broadcasted_iota(jnp.int32, (B, D), 0)
    tvec = jnp.sum(jnp.where(brow == pl.program_id(0), tvec_all, 0.0),
                   axis=0, keepdims=True)
    logits = (jnp.dot(x, wg_ref[:D], preferred_element_type=jnp.float32)
              + jnp.dot(tvec, wg_ref[D:], preferred_element_type=jnp.float32)
              + bg_ref[...])
    iota = lax.broadcasted_iota(jnp.int32, (BT, E), 1)
    m1 = jnp.max(logits, axis=-1, keepdims=True)
    a1 = jnp.min(jnp.where(logits == m1, iota, E), axis=-1, keepdims=True)
    masked = jnp.where(iota == a1, -jnp.inf, logits)
    m2 = jnp.max(masked, axis=-1, keepdims=True)
    a2 = jnp.min(jnp.where(masked == m2, iota, E), axis=-1, keepdims=True)
    g1 = 1.0 / (1.0 + jnp.exp(m2 - m1))
    g2 = 1.0 - g1
    omega = 1.0 - g1

    univ = _gelu(jnp.dot(x, wu_ref[...], preferred_element_type=jnp.float32)
                 + bu_ref[...])
    acc = omega * univ
    for e in range(E):
        w = g1 * (a1 == e) + g2 * (a2 == e)
        h = jnp.dot(x, we_ref[e], preferred_element_type=jnp.float32) + be_ref[e]
        acc = acc + w * _gelu(h)
    out_ref[0] = acc


@jax.jit
def _moe(tokens, task_onehot, task_table, Wg, bg, We, be, Wu, bu):
    grid = (B, N // BT)
    return pl.pallas_call(
        _moe_body,
        grid=grid,
        in_specs=[
            pl.BlockSpec((1, BT, D), lambda b, n: (b, n, 0)),
            pl.BlockSpec((B, T), lambda b, n: (0, 0)),
            pl.BlockSpec((T, D), lambda b, n: (0, 0)),
            pl.BlockSpec((2 * D, E), lambda b, n: (0, 0)),
            pl.BlockSpec((1, E), lambda b, n: (0, 0)),
            pl.BlockSpec((E, D, D), lambda b, n: (0, 0, 0)),
            pl.BlockSpec((E, D), lambda b, n: (0, 0)),
            pl.BlockSpec((D, D), lambda b, n: (0, 0)),
            pl.BlockSpec((1, D), lambda b, n: (0, 0)),
        ],
        out_specs=pl.BlockSpec((1, BT, D), lambda b, n: (b, n, 0)),
        out_shape=jax.ShapeDtypeStruct((B, N, D), jnp.float32),
    )(tokens, task_onehot, task_table, Wg, bg, We, be, Wu, bu)


def kernel(tokens, task_ids, task_table, Wg, bg, We, be, Wu, bu):
    task_onehot = jax.nn.one_hot(task_ids, T, dtype=jnp.float32)
    return _moe(tokens, task_onehot, task_table, Wg, bg.reshape(1, E),
                We, be, Wu, bu.reshape(1, D))


# speedup vs baseline: 4.3973x; 1.0081x over previous
import jax
import jax.numpy as jnp
from jax import lax
from jax.experimental import pallas as pl

B, N, D, E, T, K = 2, 2048, 768, 8, 16, 2
BT = 512


def _gelu(x):
    return x * 0.5 * (1.0 + lax.erf(x * (2.0 ** -0.5)))


def _moe_body(x_ref, taskoh_ref, tt_ref, wg_ref, bg_ref, we_ref, be_ref,
              wu_ref, bu_ref, out_ref):
    x = x_ref[0]
    tvec_all = jnp.dot(taskoh_ref[...], tt_ref[...],
                       preferred_element_type=jnp.float32)
    brow = lax.broadcasted_iota(jnp.int32, (B, D), 0)
    tvec = jnp.sum(jnp.where(brow == pl.program_id(0), tvec_all, 0.0),
                   axis=0, keepdims=True)
    logits = (jnp.dot(x, wg_ref[:D], preferred_element_type=jnp.float32)
              + jnp.dot(tvec, wg_ref[D:], preferred_element_type=jnp.float32)
              + bg_ref[...])
    iota = lax.broadcasted_iota(jnp.int32, (BT, E), 1)
    m1 = jnp.max(logits, axis=-1, keepdims=True)
    a1 = jnp.min(jnp.where(logits == m1, iota, E), axis=-1, keepdims=True)
    masked = jnp.where(iota == a1, -jnp.inf, logits)
    m2 = jnp.max(masked, axis=-1, keepdims=True)
    a2 = jnp.min(jnp.where(masked == m2, iota, E), axis=-1, keepdims=True)
    g1 = 1.0 / (1.0 + jnp.exp(m2 - m1))
    g2 = 1.0 - g1
    omega = 1.0 - g1

    xb = x.astype(jnp.bfloat16)
    univ = _gelu(jnp.dot(xb, wu_ref[...].astype(jnp.bfloat16),
                         preferred_element_type=jnp.float32) + bu_ref[...])
    acc = omega * univ
    for e in range(E):
        w = g1 * (a1 == e) + g2 * (a2 == e)
        h = jnp.dot(xb, we_ref[e].astype(jnp.bfloat16),
                    preferred_element_type=jnp.float32) + be_ref[e]
        acc = acc + w * _gelu(h)
    out_ref[0] = acc


@jax.jit
def _moe(tokens, task_onehot, task_table, Wg, bg, We, be, Wu, bu):
    grid = (B, N // BT)
    return pl.pallas_call(
        _moe_body,
        grid=grid,
        in_specs=[
            pl.BlockSpec((1, BT, D), lambda b, n: (b, n, 0)),
            pl.BlockSpec((B, T), lambda b, n: (0, 0)),
            pl.BlockSpec((T, D), lambda b, n: (0, 0)),
            pl.BlockSpec((2 * D, E), lambda b, n: (0, 0)),
            pl.BlockSpec((1, E), lambda b, n: (0, 0)),
            pl.BlockSpec((E, D, D), lambda b, n: (0, 0, 0)),
            pl.BlockSpec((E, D), lambda b, n: (0, 0)),
            pl.BlockSpec((D, D), lambda b, n: (0, 0)),
            pl.BlockSpec((1, D), lambda b, n: (0, 0)),
        ],
        out_specs=pl.BlockSpec((1, BT, D), lambda b, n: (b, n, 0)),
        out_shape=jax.ShapeDtypeStruct((B, N, D), jnp.float32),
    )(tokens, task_onehot, task_table, Wg, bg, We, be, Wu, bu)


def kernel(tokens, task_ids, task_table, Wg, bg, We, be, Wu, bu):
    task_onehot = jax.nn.one_hot(task_ids, T, dtype=jnp.float32)
    return _moe(tokens, task_onehot, task_table, Wg, bg.reshape(1, E),
                We, be, Wu, bu.reshape(1, D))


# speedup vs baseline: 4.5064x; 1.0248x over previous
import jax
import jax.numpy as jnp
from jax import lax
from jax.experimental import pallas as pl

B, N, D, E, T, K = 2, 2048, 768, 8, 16, 2
BT = 512


def _gelu(x):
    return x * 0.5 * (1.0 + lax.erf(x * (2.0 ** -0.5)))


def _moe_body(x_ref, taskoh_ref, tt_ref, wg_ref, bg_ref, we_ref, be_ref,
              wu_ref, bu_ref, out_ref):
    x = x_ref[0]
    tvec_all = jnp.dot(taskoh_ref[...], tt_ref[...],
                       preferred_element_type=jnp.float32)
    brow = lax.broadcasted_iota(jnp.int32, (B, D), 0)
    tvec = jnp.sum(jnp.where(brow == pl.program_id(0), tvec_all, 0.0),
                   axis=0, keepdims=True)
    logits = (jnp.dot(x, wg_ref[:D], preferred_element_type=jnp.float32)
              + jnp.dot(tvec, wg_ref[D:], preferred_element_type=jnp.float32)
              + bg_ref[...])
    iota = lax.broadcasted_iota(jnp.int32, (BT, E), 1)
    m1 = jnp.max(logits, axis=-1, keepdims=True)
    a1 = jnp.min(jnp.where(logits == m1, iota, E), axis=-1, keepdims=True)
    masked = jnp.where(iota == a1, -jnp.inf, logits)
    m2 = jnp.max(masked, axis=-1, keepdims=True)
    a2 = jnp.min(jnp.where(masked == m2, iota, E), axis=-1, keepdims=True)
    g1 = 1.0 / (1.0 + jnp.exp(m2 - m1))
    g2 = 1.0 - g1
    omega = 1.0 - g1

    xb = x.astype(jnp.bfloat16)
    h_univ = jnp.dot(xb, wu_ref[...].astype(jnp.bfloat16),
                     preferred_element_type=jnp.float32) + bu_ref[...]
    acc1 = jnp.zeros((BT, D), jnp.float32)
    acc2 = jnp.zeros((BT, D), jnp.float32)
    for e in range(E):
        h = jnp.dot(xb, we_ref[e].astype(jnp.bfloat16),
                    preferred_element_type=jnp.float32) + be_ref[e]
        acc1 = acc1 + jnp.where(a1 == e, h, 0.0)
        acc2 = acc2 + jnp.where(a2 == e, h, 0.0)
    out_ref[0] = g1 * _gelu(acc1) + g2 * _gelu(acc2) + omega * _gelu(h_univ)


@jax.jit
def _moe(tokens, task_onehot, task_table, Wg, bg, We, be, Wu, bu):
    grid = (B, N // BT)
    return pl.pallas_call(
        _moe_body,
        grid=grid,
        in_specs=[
            pl.BlockSpec((1, BT, D), lambda b, n: (b, n, 0)),
            pl.BlockSpec((B, T), lambda b, n: (0, 0)),
            pl.BlockSpec((T, D), lambda b, n: (0, 0)),
            pl.BlockSpec((2 * D, E), lambda b, n: (0, 0)),
            pl.BlockSpec((1, E), lambda b, n: (0, 0)),
            pl.BlockSpec((E, D, D), lambda b, n: (0, 0, 0)),
            pl.BlockSpec((E, D), lambda b, n: (0, 0)),
            pl.BlockSpec((D, D), lambda b, n: (0, 0)),
            pl.BlockSpec((1, D), lambda b, n: (0, 0)),
        ],
        out_specs=pl.BlockSpec((1, BT, D), lambda b, n: (b, n, 0)),
        out_shape=jax.ShapeDtypeStruct((B, N, D), jnp.float32),
    )(tokens, task_onehot, task_table, Wg, bg, We, be, Wu, bu)


def kernel(tokens, task_ids, task_table, Wg, bg, We, be, Wu, bu):
    task_onehot = jax.nn.one_hot(task_ids, T, dtype=jnp.float32)
    return _moe(tokens, task_onehot, task_table, Wg, bg.reshape(1, E),
                We, be, Wu, bu.reshape(1, D))
